# Initial kernel scaffold; baseline (speedup 1.0000x reference)
import jax
import jax.numpy as jnp
from jax.experimental import pallas as pl


def kernel(x, edge_index, batch, Wl0, bl0, Wr0, br0, Wl1, bl1, Wr1, br1, Wp1, bp1, Wp2, bp2):
    raise NotImplementedError("write your pallas kernel here")


# speedup vs baseline: 6.8079x; 6.8079x over previous
import functools

import jax
import jax.numpy as jnp
from jax import lax
from jax.experimental import pallas as pl
from jax.experimental.pallas import tpu as pltpu
from jax.experimental.pallas import tpu_sc as plsc

N = 10000
D = 128
E = 320000
NC = 2
NS = 16
NW = NC * NS
EPT = E // NW
K = 80
CH = EPT // K
RPT = 640
N_PAD = RPT * NS


def _propagate(x, er, zrows):
  mesh = plsc.VectorSubcoreMesh(core_axis_name="c", subcore_axis_name="s")

  @functools.partial(
      pl.kernel,
      out_type=jax.ShapeDtypeStruct((NC, N_PAD, D), jnp.float32),
      mesh=mesh,
      scratch_types=[
          pltpu.VMEM((CH, K), jnp.int32),
          pltpu.VMEM((CH, K), jnp.int32),
          pltpu.VMEM((K, D), jnp.float32),
          pltpu.VMEM_SHARED((N_PAD, D), jnp.float32),
          pltpu.SemaphoreType.DMA,
      ],
  )
  def prop(x_hbm, er_hbm, z_hbm, out_hbm, src_v, dst_v, rows_v, acc, sem):
    cid = lax.axis_index("c")
    sid = lax.axis_index("s")
    wid = sid * NC + cid
    pltpu.sync_copy(z_hbm, acc.at[pl.ds(sid * RPT, RPT)])
    pltpu.sync_copy(er_hbm.at[0, wid], src_v)
    pltpu.sync_copy(er_hbm.at[1, wid], dst_v)
    plsc.subcore_barrier()

    def body(j, carry):
      pltpu.async_copy(x_hbm.at[src_v.at[j]], rows_v, sem).wait()
      pltpu.sync_copy(rows_v, acc.at[dst_v.at[j]], add=True)
      return carry

    lax.fori_loop(0, CH, body, 0)
    plsc.subcore_barrier()
    pltpu.sync_copy(acc.at[pl.ds(sid * RPT, RPT)],
                    out_hbm.at[cid, pl.ds(sid * RPT, RPT)])

  return prop(x, er, zrows)


def _dotT(a, w):
  return lax.dot_general(a, w, (((1,), (1,)), ((), ())),
                         preferred_element_type=jnp.float32)


def _layer_body(x_ref, pa_ref, pb_ref, wl_ref, wr_ref, b_ref, o_ref):
  p = pa_ref[0] + pb_ref[0]
  h = _dotT(x_ref[...], wl_ref[...]) + _dotT(p, wr_ref[...]) + b_ref[...]
  nrm = jnp.sqrt(jnp.sum(h * h, axis=1, keepdims=True))
  h = h / jnp.maximum(nrm, 1e-12)
  o_ref[...] = jnp.maximum(h, 0.0)


def _final_body(x_ref, pa_ref, pb_ref, wl_ref, wr_ref, b_ref,
                wp1_ref, bp1_ref, wp2_ref, bp2_ref, o_ref):
  p = pa_ref[0] + pb_ref[0]
  h = _dotT(x_ref[...], wl_ref[...]) + _dotT(p, wr_ref[...]) + b_ref[...]
  nrm = jnp.sqrt(jnp.sum(h * h, axis=1, keepdims=True))
  h = h / jnp.maximum(nrm, 1e-12)
  h = jnp.maximum(h, 0.0)
  h = _dotT(h, wp1_ref[...]) + bp1_ref[...]
  h = _dotT(h, wp2_ref[...]) + bp2_ref[...]
  m = jnp.max(h, axis=1, keepdims=True)
  lse = m + jnp.log(jnp.sum(jnp.exp(h - m), axis=1, keepdims=True))
  o_ref[...] = h - lse


BR = 1000

_row_spec = pl.BlockSpec((BR, D), lambda i: (i, 0))
_p0_spec = pl.BlockSpec((1, BR, D), lambda i: (0, i, 0))
_p1_spec = pl.BlockSpec((1, BR, D), lambda i: (1, i, 0))
_w_spec = pl.BlockSpec((D, D), lambda i: (0, 0))
_b_spec = pl.BlockSpec((1, D), lambda i: (0, 0))


def _layer_tc(x, P, Wl, Wr, b):
  return pl.pallas_call(
      _layer_body,
      grid=(N // BR,),
      in_specs=[_row_spec, _p0_spec, _p1_spec, _w_spec, _w_spec, _b_spec],
      out_specs=_row_spec,
      out_shape=jax.ShapeDtypeStruct((N, D), jnp.float32),
  )(x, P, P, Wl, Wr, b)


def _final_tc(h, P, Wl, Wr, b, Wp1, bp1, Wp2, bp2):
  return pl.pallas_call(
      _final_body,
      grid=(N // BR,),
      in_specs=[_row_spec, _p0_spec, _p1_spec, _w_spec, _w_spec, _b_spec,
                _w_spec, _b_spec, _w_spec, _b_spec],
      out_specs=_row_spec,
      out_shape=jax.ShapeDtypeStruct((N, D), jnp.float32),
  )(h, P, P, Wl, Wr, b, Wp1, bp1, Wp2, bp2)


def kernel(x, edge_index, batch, Wl0, bl0, Wr0, br0, Wl1, bl1, Wr1, br1,
           Wp1, bp1, Wp2, bp2):
  del batch
  er = edge_index.reshape(2, NW, CH, K)
  zrows = jnp.zeros((RPT, D), jnp.float32)
  b0 = (bl0 + br0).reshape(1, D)
  b1 = (bl1 + br1).reshape(1, D)

  P0 = _propagate(x, er, zrows)
  h = _layer_tc(x, P0, Wl0, Wr0, b0)
  P1 = _propagate(h, er, zrows)
  return _final_tc(h, P1, Wl1, Wr1, b1, Wp1, bp1.reshape(1, D),
                   Wp2, bp2.reshape(1, D))
